# baseline (speedup 1.0000x reference)
import functools

import jax
import jax.numpy as jnp
from jax import lax
from jax.experimental import pallas as pl
from jax.experimental.pallas import tpu as pltpu
from jax.experimental.pallas import tpu_sc as plsc

E = 8
K = 2
D = 1024
F = 1024
T = 8192
S = T * K
BLK = 256
NB = S // BLK + E
P = NB * BLK
TBLK = 1024

NC = 2
NS = 16
NW = NC * NS
TPW = T // NW
CS = 32
C2 = 16

_INTERPRET = False


def _gating_body(x_ref, gw_ref, gb_ref, i0_ref, i1_ref, s0_ref, s1_ref):
    x = x_ref[...]
    logits = jax.lax.dot_general(
        x, gw_ref[...], (((1,), (0,)), ((), ())),
        preferred_element_type=jnp.float32) + gb_ref[...]
    e_iota = jax.lax.broadcasted_iota(jnp.int32, logits.shape, 1)
    i0 = jnp.argmax(logits, axis=1).astype(jnp.int32)
    v0 = jnp.max(logits, axis=1)
    masked = jnp.where(e_iota == i0[:, None], -jnp.inf, logits)
    i1 = jnp.argmax(masked, axis=1).astype(jnp.int32)
    v1 = jnp.max(masked, axis=1)
    p0 = 1.0 / (1.0 + jnp.exp(v1 - v0))
    i0_ref[0, :] = i0
    i1_ref[0, :] = i1
    s0_ref[0, :] = p0
    s1_ref[0, :] = 1.0 - p0


def _gating(moe_inp, gate_w, gate_b):
    out_shape = [
        jax.ShapeDtypeStruct((1, T), jnp.int32),
        jax.ShapeDtypeStruct((1, T), jnp.int32),
        jax.ShapeDtypeStruct((1, T), jnp.float32),
        jax.ShapeDtypeStruct((1, T), jnp.float32),
    ]
    spec1t = pl.BlockSpec((1, TBLK), lambda b: (0, b))
    return pl.pallas_call(
        _gating_body,
        grid=(T // TBLK,),
        in_specs=[
            pl.BlockSpec((TBLK, D), lambda b: (b, 0)),
            pl.BlockSpec((D, E), lambda b: (0, 0)),
            pl.BlockSpec((1, E), lambda b: (0, 0)),
        ],
        out_specs=[spec1t, spec1t, spec1t, spec1t],
        out_shape=out_shape,
        interpret=_INTERPRET,
    )(moe_inp, gate_w, gate_b.reshape(1, E))


def _mlp_body(be_ref, x_ref, w1_ref, b1_ref, w2_ref, b2_ref, y_ref):
    x = x_ref[...].astype(jnp.bfloat16)
    h = jax.lax.dot_general(x, w1_ref[0].astype(jnp.bfloat16),
                            (((1,), (0,)), ((), ())),
                            preferred_element_type=jnp.float32) + b1_ref[0]
    h = jnp.maximum(h, 0.0).astype(jnp.bfloat16)
    y_ref[...] = jax.lax.dot_general(h, w2_ref[0].astype(jnp.bfloat16),
                                     (((1,), (0,)), ((), ())),
                                     preferred_element_type=jnp.float32) + b2_ref[0]


def _grouped_mlp(block_expert, x_pad, w1, b1, w2, b2):
    grid_spec = pltpu.PrefetchScalarGridSpec(
        num_scalar_prefetch=1,
        grid=(NB,),
        in_specs=[
            pl.BlockSpec((BLK, D), lambda b, be: (b, 0)),
            pl.BlockSpec((1, D, F), lambda b, be: (be[b], 0, 0)),
            pl.BlockSpec((1, 1, F), lambda b, be: (be[b], 0, 0)),
            pl.BlockSpec((1, F, D), lambda b, be: (be[b], 0, 0)),
            pl.BlockSpec((1, 1, D), lambda b, be: (be[b], 0, 0)),
        ],
        out_specs=pl.BlockSpec((BLK, D), lambda b, be: (b, 0)),
    )
    return pl.pallas_call(
        _mlp_body,
        grid_spec=grid_spec,
        out_shape=jax.ShapeDtypeStruct((P, D), jnp.float32),
        interpret=_INTERPRET,
    )(block_expert, x_pad, w1, b1.reshape(E, 1, F), w2, b2.reshape(E, 1, D))


NB_PAD = 80
TPT = T // NS


def _sc_route_scatter(i0, i1, s0, s1, moe_inp):
    mesh = plsc.VectorSubcoreMesh(core_axis_name="c", subcore_axis_name="s")

    @functools.partial(
        pl.kernel, mesh=mesh,
        out_type=[
            jax.ShapeDtypeStruct((P, D), jnp.float32),
            jax.ShapeDtypeStruct((T,), jnp.int32),
            jax.ShapeDtypeStruct((T,), jnp.int32),
            jax.ShapeDtypeStruct((NB_PAD,), jnp.int32),
        ],
        scratch_types=[
            pltpu.VMEM((TPT,), jnp.int32),
            pltpu.VMEM((TPT,), jnp.int32),
            pltpu.VMEM((TPT,), jnp.int32),
            pltpu.VMEM((TPT,), jnp.int32),
            pltpu.VMEM((16,), jnp.int32),
            pltpu.VMEM((NS, 16), jnp.int32),
            pltpu.VMEM_SHARED((NS, 16), jnp.int32),
            pltpu.VMEM((NB_PAD,), jnp.int32),
            pltpu.VMEM((TPW,), jnp.float32),
            pltpu.VMEM((TPW,), jnp.float32),
            pltpu.VMEM((CS, D), jnp.float32),
            pltpu.VMEM((CS, D), jnp.float32),
            pltpu.VMEM((CS, D), jnp.float32),
            pltpu.SemaphoreType.DMA,
            pltpu.SemaphoreType.DMA,
        ],
    )
    def k(i0_hbm, i1_hbm, s0_hbm, s1_hbm, x_hbm,
          xp_hbm, d0_hbm, d1_hbm, be_hbm,
          e0_v, e1_v, r0_v, r1_v, cnt_v, grid_v, grid_sh, be_v,
          s0_v, s1_v, rows_v, b0_v, b1_v, sem0, sem1):
        cid = lax.axis_index("c")
        sid = lax.axis_index("s")

        base = sid * TPT
        pltpu.sync_copy(i0_hbm.at[pl.ds(base, TPT)], e0_v)
        pltpu.sync_copy(i1_hbm.at[pl.ds(base, TPT)], e1_v)

        def count_half(e_ref, r_ref, counts):
            def body(j, counts):
                ev = e_ref[pl.ds(j * 16, 16)]
                rank = jnp.zeros((16,), jnp.int32)
                new = []
                for e in range(E):
                    m = ev == e
                    rank = rank + jnp.where(m, counts[e], 0)
                    new.append(counts[e] + jnp.where(m, 1, 0))
                r_ref[pl.ds(j * 16, 16)] = rank
                return tuple(new)
            return lax.fori_loop(0, TPT // 16, body, counts)

        counts = tuple(jnp.zeros((16,), jnp.int32) for _ in range(E))
        counts = count_half(e0_v, r0_v, counts)
        counts = count_half(e1_v, r1_v, counts)

        lane = lax.iota(jnp.int32, 16)
        lane_base = []
        tot_vec = jnp.zeros((16,), jnp.int32)
        for e in range(E):
            acc = jnp.int32(0)
            lb = jnp.zeros((16,), jnp.int32)
            for l in range(16):
                lb = jnp.where(lane == l, acc, lb)
                acc = acc + counts[e][l]
            lane_base.append(lb)
            tot_vec = jnp.where(lane == e, acc, tot_vec)
        cnt_v[...] = tot_vec

        pltpu.sync_copy(cnt_v, grid_sh.at[sid])
        plsc.subcore_barrier()
        pltpu.sync_copy(grid_sh, grid_v)

        tile_base = [jnp.int32(0)] * E
        tot = [jnp.int32(0)] * E
        for t in range(NS):
            row = grid_v[t]
            for e in range(E):
                v = row[e]
                tile_base[e] = tile_base[e] + jnp.where(sid > t, v, 0)
                tot[e] = tot[e] + v

        seg = []
        bstart = []
        acc = jnp.int32(0)
        for e in range(E):
            bstart.append(acc)
            seg.append(acc * BLK)
            acc = acc + (tot[e] + (BLK - 1)) // BLK

        def dest_half(e_ref, r_ref, d_hbm):
            def body(j, _):
                ev = e_ref[pl.ds(j * 16, 16)]
                dv = r_ref[pl.ds(j * 16, 16)]
                for e in range(E):
                    dv = dv + jnp.where(
                        ev == e, lane_base[e] + (seg[e] + tile_base[e]), 0)
                r_ref[pl.ds(j * 16, 16)] = dv
                return 0
            lax.fori_loop(0, TPT // 16, body, 0)

            @pl.when(cid == 0)
            def _wr():
                pltpu.sync_copy(r_ref, d_hbm.at[pl.ds(base, TPT)])

        dest_half(e0_v, r0_v, d0_hbm)
        dest_half(e1_v, r1_v, d1_hbm)

        @pl.when(jnp.logical_and(sid == 0, cid == 0))
        def _be():
            for j in range(NB_PAD // 16):
                bvec = lax.iota(jnp.int32, 16) + (j * 16)
                be = jnp.zeros((16,), jnp.int32)
                for e in range(1, E):
                    be = be + jnp.where(bvec >= bstart[e], 1, 0)
                be_v[pl.ds(j * 16, 16)] = be
            pltpu.sync_copy(be_v, be_hbm)

        tbase = base + cid * TPW
        off = cid * TPW
        pltpu.sync_copy(s0_hbm.at[pl.ds(tbase, TPW)], s0_v)
        pltpu.sync_copy(s1_hbm.at[pl.ds(tbase, TPW)], s1_v)

        def chunk_body(c, _):
            pltpu.sync_copy(x_hbm.at[pl.ds(tbase + c * CS, CS)], rows_v)
            for g in range(CS // 16):
                svec0 = s0_v[pl.ds(c * CS + g * 16, 16)]
                svec1 = s1_v[pl.ds(c * CS + g * 16, 16)]
                for r16 in range(16):
                    r = g * 16 + r16
                    sb0 = svec0[r16]
                    sb1 = svec1[r16]

                    def jbody(j, _, r=r, sb0=sb0, sb1=sb1):
                        sl = pl.ds(j * 16, 16)
                        rv = rows_v[r, sl]
                        b0_v[r, sl] = sb0 * rv
                        b1_v[r, sl] = sb1 * rv
                        return 0

                    lax.fori_loop(0, D // 16, jbody, 0, unroll=8)
            a = pltpu.async_copy(
                b0_v, xp_hbm.at[r0_v.at[pl.ds(off + c * CS, CS)]], sem0)
            b = pltpu.async_copy(
                b1_v, xp_hbm.at[r1_v.at[pl.ds(off + c * CS, CS)]], sem1)
            a.wait()
            b.wait()
            return 0

        lax.fori_loop(0, TPW // CS, chunk_body, 0)

    return k(i0, i1, s0, s1, moe_inp)


def _sc_combine(y_pad, dest0, dest1):
    mesh = plsc.VectorSubcoreMesh(core_axis_name="c", subcore_axis_name="s")
    nchunk = TPW // C2

    @functools.partial(
        pl.kernel, mesh=mesh,
        out_type=jax.ShapeDtypeStruct((T, D), jnp.float32),
        scratch_types=[
            pltpu.VMEM((TPW,), jnp.int32),
            pltpu.VMEM((TPW,), jnp.int32),
            pltpu.VMEM((2, C2, D), jnp.float32),
            pltpu.VMEM((2, C2, D), jnp.float32),
            pltpu.SemaphoreType.DMA,
            pltpu.SemaphoreType.DMA,
            pltpu.SemaphoreType.DMA,
        ],
    )
    def k(y_hbm, d0_hbm, d1_hbm, out_hbm, i0_v, i1_v, buf0, buf1,
          gsem0, gsem1, osem):
        wid = lax.axis_index("s") * NC + lax.axis_index("c")
        base = wid * TPW
        pltpu.sync_copy(d0_hbm.at[pl.ds(base, TPW)], i0_v)
        pltpu.sync_copy(d1_hbm.at[pl.ds(base, TPW)], i1_v)

        def gather(c, slot):
            pltpu.async_copy(y_hbm.at[i0_v.at[pl.ds(c * C2, C2)]],
                             buf0.at[slot], gsem0)
            pltpu.async_copy(y_hbm.at[i1_v.at[pl.ds(c * C2, C2)]],
                             buf1.at[slot], gsem1)

        def wait_gather(c, slot):
            pltpu.make_async_copy(y_hbm.at[i0_v.at[pl.ds(c * C2, C2)]],
                                  buf0.at[slot], gsem0).wait()
            pltpu.make_async_copy(y_hbm.at[i1_v.at[pl.ds(c * C2, C2)]],
                                  buf1.at[slot], gsem1).wait()

        def wait_out(c, slot):
            pltpu.make_async_copy(
                buf0.at[slot],
                out_hbm.at[pl.ds(base + c * C2, C2)], osem).wait()

        gather(0, 0)

        def chunk_body(c, _):
            slot = lax.rem(c, 2)
            wait_gather(c, slot)

            @pl.when(c + 1 < nchunk)
            def _pref():
                gather(c + 1, lax.rem(c + 1, 2))

            @pl.when(c >= 2)
            def _owait():
                wait_out(c - 2, slot)

            for r in range(C2):

                def jbody(j, _, r=r):
                    sl = pl.ds(j * 16, 16)
                    buf0[slot, r, sl] = buf0[slot, r, sl] + buf1[slot, r, sl]
                    return 0

                lax.fori_loop(0, D // 16, jbody, 0, unroll=8)
            pltpu.async_copy(buf0.at[slot],
                             out_hbm.at[pl.ds(base + c * C2, C2)], osem)
            return 0

        lax.fori_loop(0, nchunk, chunk_body, 0)
        wait_out(nchunk - 2, lax.rem(nchunk - 2, 2))
        wait_out(nchunk - 1, lax.rem(nchunk - 1, 2))

    return k(y_pad, dest0, dest1)


def kernel(moe_inp, attn_weights, gate_w, gate_b, w1, b1, w2, b2):
    del attn_weights
    i0, i1, s0, s1 = _gating(moe_inp, gate_w, gate_b)
    i0, i1, s0, s1 = i0[0], i1[0], s0[0], s1[0]

    x_pad, dest0, dest1, block_expert = _sc_route_scatter(
        i0, i1, s0, s1, moe_inp)
    y_pad = _grouped_mlp(block_expert, x_pad, w1, b1, w2, b2)
    out = _sc_combine(y_pad, dest0, dest1)
    return out


# speedup vs baseline: 1.1086x; 1.1086x over previous
import functools

import jax
import jax.numpy as jnp
from jax import lax
from jax.experimental import pallas as pl
from jax.experimental.pallas import tpu as pltpu
from jax.experimental.pallas import tpu_sc as plsc

E = 8
K = 2
D = 1024
F = 1024
T = 8192
S = T * K
BLK = 256
NB = S // BLK + E
P = NB * BLK
TBLK = 1024

NC = 2
NS = 16
NW = NC * NS
TPW = T // NW
CS = 32
C2 = 16

_INTERPRET = False


def _gating_body(x_ref, gw_ref, gb_ref, i0_ref, i1_ref, s0_ref, s1_ref):
    x = x_ref[...]
    logits = jax.lax.dot_general(
        x, gw_ref[...], (((1,), (0,)), ((), ())),
        preferred_element_type=jnp.float32) + gb_ref[...]
    e_iota = jax.lax.broadcasted_iota(jnp.int32, logits.shape, 1)
    i0 = jnp.argmax(logits, axis=1).astype(jnp.int32)
    v0 = jnp.max(logits, axis=1)
    masked = jnp.where(e_iota == i0[:, None], -jnp.inf, logits)
    i1 = jnp.argmax(masked, axis=1).astype(jnp.int32)
    v1 = jnp.max(masked, axis=1)
    p0 = 1.0 / (1.0 + jnp.exp(v1 - v0))
    i0_ref[0, :] = i0
    i1_ref[0, :] = i1
    s0_ref[0, :] = p0
    s1_ref[0, :] = 1.0 - p0


def _gating(moe_inp, gate_w, gate_b):
    out_shape = [
        jax.ShapeDtypeStruct((1, T), jnp.int32),
        jax.ShapeDtypeStruct((1, T), jnp.int32),
        jax.ShapeDtypeStruct((1, T), jnp.float32),
        jax.ShapeDtypeStruct((1, T), jnp.float32),
    ]
    spec1t = pl.BlockSpec((1, TBLK), lambda b: (0, b))
    return pl.pallas_call(
        _gating_body,
        grid=(T // TBLK,),
        in_specs=[
            pl.BlockSpec((TBLK, D), lambda b: (b, 0)),
            pl.BlockSpec((D, E), lambda b: (0, 0)),
            pl.BlockSpec((1, E), lambda b: (0, 0)),
        ],
        out_specs=[spec1t, spec1t, spec1t, spec1t],
        out_shape=out_shape,
        interpret=_INTERPRET,
    )(moe_inp, gate_w, gate_b.reshape(1, E))


def _mlp_body(be_ref, x_ref, w1_ref, b1_ref, w2_ref, b2_ref, y_ref):
    x = x_ref[...].astype(jnp.bfloat16)
    h = jax.lax.dot_general(x, w1_ref[0].astype(jnp.bfloat16),
                            (((1,), (0,)), ((), ())),
                            preferred_element_type=jnp.float32) + b1_ref[0]
    h = jnp.maximum(h, 0.0).astype(jnp.bfloat16)
    y_ref[...] = jax.lax.dot_general(h, w2_ref[0].astype(jnp.bfloat16),
                                     (((1,), (0,)), ((), ())),
                                     preferred_element_type=jnp.float32) + b2_ref[0]


def _grouped_mlp(block_expert, x_pad, w1, b1, w2, b2):
    grid_spec = pltpu.PrefetchScalarGridSpec(
        num_scalar_prefetch=1,
        grid=(NB,),
        in_specs=[
            pl.BlockSpec((BLK, D), lambda b, be: (b, 0)),
            pl.BlockSpec((1, D, F), lambda b, be: (be[b], 0, 0)),
            pl.BlockSpec((1, 1, F), lambda b, be: (be[b], 0, 0)),
            pl.BlockSpec((1, F, D), lambda b, be: (be[b], 0, 0)),
            pl.BlockSpec((1, 1, D), lambda b, be: (be[b], 0, 0)),
        ],
        out_specs=pl.BlockSpec((BLK, D), lambda b, be: (b, 0)),
    )
    return pl.pallas_call(
        _mlp_body,
        grid_spec=grid_spec,
        out_shape=jax.ShapeDtypeStruct((P, D), jnp.float32),
        interpret=_INTERPRET,
    )(block_expert, x_pad, w1, b1.reshape(E, 1, F), w2, b2.reshape(E, 1, D))


NB_PAD = 80
TPT = T // NS


def _sc_route_scatter(i0, i1, moe_inp):
    mesh = plsc.VectorSubcoreMesh(core_axis_name="c", subcore_axis_name="s")

    @functools.partial(
        pl.kernel, mesh=mesh,
        out_type=[
            jax.ShapeDtypeStruct((P, D), jnp.float32),
            jax.ShapeDtypeStruct((T,), jnp.int32),
            jax.ShapeDtypeStruct((T,), jnp.int32),
            jax.ShapeDtypeStruct((NB_PAD,), jnp.int32),
        ],
        scratch_types=[
            pltpu.VMEM((TPT,), jnp.int32),
            pltpu.VMEM((TPT,), jnp.int32),
            pltpu.VMEM((TPT,), jnp.int32),
            pltpu.VMEM((TPT,), jnp.int32),
            pltpu.VMEM((16,), jnp.int32),
            pltpu.VMEM((NS, 16), jnp.int32),
            pltpu.VMEM_SHARED((NS, 16), jnp.int32),
            pltpu.VMEM((NB_PAD,), jnp.int32),
            pltpu.VMEM((2, CS, D), jnp.float32),
            pltpu.SemaphoreType.DMA,
            pltpu.SemaphoreType.DMA,
        ],
    )
    def k(i0_hbm, i1_hbm, x_hbm,
          xp_hbm, d0_hbm, d1_hbm, be_hbm,
          e0_v, e1_v, r0_v, r1_v, cnt_v, grid_v, grid_sh, be_v,
          rows2_v, sem0, sem1):
        cid = lax.axis_index("c")
        sid = lax.axis_index("s")

        base = sid * TPT
        pltpu.sync_copy(i0_hbm.at[pl.ds(base, TPT)], e0_v)
        pltpu.sync_copy(i1_hbm.at[pl.ds(base, TPT)], e1_v)

        def count_half(e_ref, r_ref, counts):
            def body(j, counts):
                ev = e_ref[pl.ds(j * 16, 16)]
                rank = jnp.zeros((16,), jnp.int32)
                new = []
                for e in range(E):
                    m = ev == e
                    rank = rank + jnp.where(m, counts[e], 0)
                    new.append(counts[e] + jnp.where(m, 1, 0))
                r_ref[pl.ds(j * 16, 16)] = rank
                return tuple(new)
            return lax.fori_loop(0, TPT // 16, body, counts)

        counts = tuple(jnp.zeros((16,), jnp.int32) for _ in range(E))
        counts = count_half(e0_v, r0_v, counts)
        counts = count_half(e1_v, r1_v, counts)

        lane = lax.iota(jnp.int32, 16)
        lane_base = []
        tot_vec = jnp.zeros((16,), jnp.int32)
        for e in range(E):
            acc = jnp.int32(0)
            lb = jnp.zeros((16,), jnp.int32)
            for l in range(16):
                lb = jnp.where(lane == l, acc, lb)
                acc = acc + counts[e][l]
            lane_base.append(lb)
            tot_vec = jnp.where(lane == e, acc, tot_vec)
        cnt_v[...] = tot_vec

        pltpu.sync_copy(cnt_v, grid_sh.at[sid])
        plsc.subcore_barrier()
        pltpu.sync_copy(grid_sh, grid_v)

        tile_base = [jnp.int32(0)] * E
        tot = [jnp.int32(0)] * E
        for t in range(NS):
            row = grid_v[t]
            for e in range(E):
                v = row[e]
                tile_base[e] = tile_base[e] + jnp.where(sid > t, v, 0)
                tot[e] = tot[e] + v

        seg = []
        bstart = []
        acc = jnp.int32(0)
        for e in range(E):
            bstart.append(acc)
            seg.append(acc * BLK)
            acc = acc + (tot[e] + (BLK - 1)) // BLK

        def dest_half(e_ref, r_ref, d_hbm):
            def body(j, _):
                ev = e_ref[pl.ds(j * 16, 16)]
                dv = r_ref[pl.ds(j * 16, 16)]
                for e in range(E):
                    dv = dv + jnp.where(
                        ev == e, lane_base[e] + (seg[e] + tile_base[e]), 0)
                r_ref[pl.ds(j * 16, 16)] = dv
                return 0
            lax.fori_loop(0, TPT // 16, body, 0)

            @pl.when(cid == 0)
            def _wr():
                pltpu.sync_copy(r_ref, d_hbm.at[pl.ds(base, TPT)])

        dest_half(e0_v, r0_v, d0_hbm)
        dest_half(e1_v, r1_v, d1_hbm)

        @pl.when(jnp.logical_and(sid == 0, cid == 0))
        def _be():
            for j in range(NB_PAD // 16):
                bvec = lax.iota(jnp.int32, 16) + (j * 16)
                be = jnp.zeros((16,), jnp.int32)
                for e in range(1, E):
                    be = be + jnp.where(bvec >= bstart[e], 1, 0)
                be_v[pl.ds(j * 16, 16)] = be
            pltpu.sync_copy(be_v, be_hbm)

        tbase = base + cid * TPW
        off = cid * TPW

        def chunk_body(c, _):
            slot = lax.rem(c, 2)
            pltpu.sync_copy(x_hbm.at[pl.ds(tbase + c * CS, CS)],
                            rows2_v.at[slot])
            a = pltpu.async_copy(
                rows2_v.at[slot],
                xp_hbm.at[r0_v.at[pl.ds(off + c * CS, CS)]], sem0)
            b = pltpu.async_copy(
                rows2_v.at[slot],
                xp_hbm.at[r1_v.at[pl.ds(off + c * CS, CS)]], sem1)
            a.wait()
            b.wait()
            return 0

        lax.fori_loop(0, TPW // CS, chunk_body, 0)

    return k(i0, i1, moe_inp)


def _sc_combine(y_pad, dest0, dest1, s0, s1):
    mesh = plsc.VectorSubcoreMesh(core_axis_name="c", subcore_axis_name="s")

    nchunk = TPW // C2

    @functools.partial(
        pl.kernel, mesh=mesh,
        out_type=jax.ShapeDtypeStruct((T, D), jnp.float32),
        scratch_types=[
            pltpu.VMEM((TPW,), jnp.int32),
            pltpu.VMEM((TPW,), jnp.int32),
            pltpu.VMEM((TPW,), jnp.float32),
            pltpu.VMEM((TPW,), jnp.float32),
            pltpu.VMEM((2, C2, D), jnp.float32),
            pltpu.VMEM((2, C2, D), jnp.float32),
            pltpu.SemaphoreType.DMA,
            pltpu.SemaphoreType.DMA,
            pltpu.SemaphoreType.DMA,
        ],
    )
    def k(y_hbm, d0_hbm, d1_hbm, s0_hbm, s1_hbm, out_hbm,
          i0_v, i1_v, s0_v, s1_v, buf0, buf1, gsem0, gsem1, osem):
        wid = lax.axis_index("s") * NC + lax.axis_index("c")
        base = wid * TPW
        pltpu.sync_copy(d0_hbm.at[pl.ds(base, TPW)], i0_v)
        pltpu.sync_copy(d1_hbm.at[pl.ds(base, TPW)], i1_v)
        pltpu.sync_copy(s0_hbm.at[pl.ds(base, TPW)], s0_v)
        pltpu.sync_copy(s1_hbm.at[pl.ds(base, TPW)], s1_v)

        def gather(c, slot):
            pltpu.async_copy(y_hbm.at[i0_v.at[pl.ds(c * C2, C2)]],
                             buf0.at[slot], gsem0)
            pltpu.async_copy(y_hbm.at[i1_v.at[pl.ds(c * C2, C2)]],
                             buf1.at[slot], gsem1)

        def wait_gather(c, slot):
            pltpu.make_async_copy(y_hbm.at[i0_v.at[pl.ds(c * C2, C2)]],
                                  buf0.at[slot], gsem0).wait()
            pltpu.make_async_copy(y_hbm.at[i1_v.at[pl.ds(c * C2, C2)]],
                                  buf1.at[slot], gsem1).wait()

        def wait_out(c, slot):
            pltpu.make_async_copy(
                buf0.at[slot],
                out_hbm.at[pl.ds(base + c * C2, C2)], osem).wait()

        gather(0, 0)

        def chunk_body(c, _):
            slot = lax.rem(c, 2)
            wait_gather(c, slot)

            @pl.when(c + 1 < nchunk)
            def _pref():
                gather(c + 1, lax.rem(c + 1, 2))

            @pl.when(c >= 2)
            def _owait():
                wait_out(c - 2, slot)

            for g in range(C2 // 16):
                svec0 = s0_v[pl.ds(c * C2 + g * 16, 16)]
                svec1 = s1_v[pl.ds(c * C2 + g * 16, 16)]
                for r16 in range(16):
                    r = g * 16 + r16
                    sb0 = svec0[r16]
                    sb1 = svec1[r16]

                    def jbody(j, _, r=r, sb0=sb0, sb1=sb1):
                        sl = pl.ds(j * 16, 16)
                        buf0[slot, r, sl] = (sb0 * buf0[slot, r, sl]
                                             + sb1 * buf1[slot, r, sl])
                        return 0

                    lax.fori_loop(0, D // 16, jbody, 0, unroll=8)
            pltpu.async_copy(buf0.at[slot],
                             out_hbm.at[pl.ds(base + c * C2, C2)], osem)
            return 0

        lax.fori_loop(0, nchunk, chunk_body, 0)
        wait_out(nchunk - 2, lax.rem(nchunk - 2, 2))
        wait_out(nchunk - 1, lax.rem(nchunk - 1, 2))

    return k(y_pad, dest0, dest1, s0, s1)


def kernel(moe_inp, attn_weights, gate_w, gate_b, w1, b1, w2, b2):
    del attn_weights
    i0, i1, s0, s1 = _gating(moe_inp, gate_w, gate_b)
    i0, i1, s0, s1 = i0[0], i1[0], s0[0], s1[0]

    x_pad, dest0, dest1, block_expert = _sc_route_scatter(i0, i1, moe_inp)
    y_pad = _grouped_mlp(block_expert, x_pad, w1, b1, w2, b2)
    out = _sc_combine(y_pad, dest0, dest1, s0, s1)
    return out


# speedup vs baseline: 1.1087x; 1.0001x over previous
import functools

import jax
import jax.numpy as jnp
from jax import lax
from jax.experimental import pallas as pl
from jax.experimental.pallas import tpu as pltpu
from jax.experimental.pallas import tpu_sc as plsc

E = 8
K = 2
D = 1024
F = 1024
T = 8192
S = T * K
BLK = 256
NB = S // BLK + E
P = NB * BLK
TBLK = 1024

NC = 2
NS = 16
NW = NC * NS
TPW = T // NW
CS = 32
C2 = 16


def _gating_body(x_ref, gw_ref, gb_ref, i0_ref, i1_ref, s0_ref, s1_ref):
    x = x_ref[...]
    logits = jax.lax.dot_general(
        x, gw_ref[...], (((1,), (0,)), ((), ())),
        preferred_element_type=jnp.float32) + gb_ref[...]
    e_iota = jax.lax.broadcasted_iota(jnp.int32, logits.shape, 1)
    i0 = jnp.argmax(logits, axis=1).astype(jnp.int32)
    v0 = jnp.max(logits, axis=1)
    masked = jnp.where(e_iota == i0[:, None], -jnp.inf, logits)
    i1 = jnp.argmax(masked, axis=1).astype(jnp.int32)
    v1 = jnp.max(masked, axis=1)
    p0 = 1.0 / (1.0 + jnp.exp(v1 - v0))
    i0_ref[0, :] = i0
    i1_ref[0, :] = i1
    s0_ref[0, :] = p0
    s1_ref[0, :] = 1.0 - p0


def _gating(moe_inp, gate_w, gate_b):
    out_shape = [
        jax.ShapeDtypeStruct((1, T), jnp.int32),
        jax.ShapeDtypeStruct((1, T), jnp.int32),
        jax.ShapeDtypeStruct((1, T), jnp.float32),
        jax.ShapeDtypeStruct((1, T), jnp.float32),
    ]
    spec1t = pl.BlockSpec((1, TBLK), lambda b: (0, b))
    return pl.pallas_call(
        _gating_body,
        grid=(T // TBLK,),
        in_specs=[
            pl.BlockSpec((TBLK, D), lambda b: (b, 0)),
            pl.BlockSpec((D, E), lambda b: (0, 0)),
            pl.BlockSpec((1, E), lambda b: (0, 0)),
        ],
        out_specs=[spec1t, spec1t, spec1t, spec1t],
        out_shape=out_shape,
    )(moe_inp, gate_w, gate_b.reshape(1, E))


def _mlp_body(be_ref, x_ref, w1_ref, b1_ref, w2_ref, b2_ref, y_ref):
    x = x_ref[...].astype(jnp.bfloat16)
    h = jax.lax.dot_general(x, w1_ref[0].astype(jnp.bfloat16),
                            (((1,), (0,)), ((), ())),
                            preferred_element_type=jnp.float32) + b1_ref[0]
    h = jnp.maximum(h, 0.0).astype(jnp.bfloat16)
    y_ref[...] = jax.lax.dot_general(h, w2_ref[0].astype(jnp.bfloat16),
                                     (((1,), (0,)), ((), ())),
                                     preferred_element_type=jnp.float32) + b2_ref[0]


def _grouped_mlp(block_expert, x_pad, w1, b1, w2, b2):
    grid_spec = pltpu.PrefetchScalarGridSpec(
        num_scalar_prefetch=1,
        grid=(NB,),
        in_specs=[
            pl.BlockSpec((BLK, D), lambda b, be: (b, 0)),
            pl.BlockSpec((1, D, F), lambda b, be: (be[b], 0, 0)),
            pl.BlockSpec((1, 1, F), lambda b, be: (be[b], 0, 0)),
            pl.BlockSpec((1, F, D), lambda b, be: (be[b], 0, 0)),
            pl.BlockSpec((1, 1, D), lambda b, be: (be[b], 0, 0)),
        ],
        out_specs=pl.BlockSpec((BLK, D), lambda b, be: (b, 0)),
    )
    return pl.pallas_call(
        _mlp_body,
        grid_spec=grid_spec,
        out_shape=jax.ShapeDtypeStruct((P, D), jnp.float32),
    )(block_expert, x_pad, w1, b1.reshape(E, 1, F), w2, b2.reshape(E, 1, D))


NB_PAD = 80
TPT = T // NS


def _sc_route_scatter(i0, i1, moe_inp):
    mesh = plsc.VectorSubcoreMesh(core_axis_name="c", subcore_axis_name="s")

    @functools.partial(
        pl.kernel, mesh=mesh,
        out_type=[
            jax.ShapeDtypeStruct((P, D), jnp.float32),
            jax.ShapeDtypeStruct((T,), jnp.int32),
            jax.ShapeDtypeStruct((T,), jnp.int32),
            jax.ShapeDtypeStruct((NB_PAD,), jnp.int32),
        ],
        scratch_types=[
            pltpu.VMEM((TPT,), jnp.int32),
            pltpu.VMEM((TPT,), jnp.int32),
            pltpu.VMEM((TPT,), jnp.int32),
            pltpu.VMEM((TPT,), jnp.int32),
            pltpu.VMEM((16,), jnp.int32),
            pltpu.VMEM((NS, 16), jnp.int32),
            pltpu.VMEM_SHARED((NS, 16), jnp.int32),
            pltpu.VMEM((NB_PAD,), jnp.int32),
            pltpu.VMEM((2, CS, D), jnp.float32),
            pltpu.SemaphoreType.DMA,
            pltpu.SemaphoreType.DMA,
        ],
    )
    def k(i0_hbm, i1_hbm, x_hbm,
          xp_hbm, d0_hbm, d1_hbm, be_hbm,
          e0_v, e1_v, r0_v, r1_v, cnt_v, grid_v, grid_sh, be_v,
          rows2_v, sem0, sem1):
        cid = lax.axis_index("c")
        sid = lax.axis_index("s")

        base = sid * TPT
        pltpu.sync_copy(i0_hbm.at[pl.ds(base, TPT)], e0_v)
        pltpu.sync_copy(i1_hbm.at[pl.ds(base, TPT)], e1_v)

        def count_half(e_ref, r_ref, counts):
            def body(j, counts):
                ev = e_ref[pl.ds(j * 16, 16)]
                rank = jnp.zeros((16,), jnp.int32)
                new = []
                for e in range(E):
                    m = ev == e
                    rank = rank + jnp.where(m, counts[e], 0)
                    new.append(counts[e] + jnp.where(m, 1, 0))
                r_ref[pl.ds(j * 16, 16)] = rank
                return tuple(new)
            return lax.fori_loop(0, TPT // 16, body, counts)

        counts = tuple(jnp.zeros((16,), jnp.int32) for _ in range(E))
        counts = count_half(e0_v, r0_v, counts)
        counts = count_half(e1_v, r1_v, counts)

        lane = lax.iota(jnp.int32, 16)
        lane_base = []
        tot_vec = jnp.zeros((16,), jnp.int32)
        for e in range(E):
            acc = jnp.int32(0)
            lb = jnp.zeros((16,), jnp.int32)
            for l in range(16):
                lb = jnp.where(lane == l, acc, lb)
                acc = acc + counts[e][l]
            lane_base.append(lb)
            tot_vec = jnp.where(lane == e, acc, tot_vec)
        cnt_v[...] = tot_vec

        pltpu.sync_copy(cnt_v, grid_sh.at[sid])
        plsc.subcore_barrier()
        pltpu.sync_copy(grid_sh, grid_v)

        tile_base = [jnp.int32(0)] * E
        tot = [jnp.int32(0)] * E
        for t in range(NS):
            row = grid_v[t]
            for e in range(E):
                v = row[e]
                tile_base[e] = tile_base[e] + jnp.where(sid > t, v, 0)
                tot[e] = tot[e] + v

        seg = []
        bstart = []
        acc = jnp.int32(0)
        for e in range(E):
            bstart.append(acc)
            seg.append(acc * BLK)
            acc = acc + (tot[e] + (BLK - 1)) // BLK

        def dest_half(e_ref, r_ref, d_hbm):
            def body(j, _):
                ev = e_ref[pl.ds(j * 16, 16)]
                dv = r_ref[pl.ds(j * 16, 16)]
                for e in range(E):
                    dv = dv + jnp.where(
                        ev == e, lane_base[e] + (seg[e] + tile_base[e]), 0)
                r_ref[pl.ds(j * 16, 16)] = dv
                return 0
            lax.fori_loop(0, TPT // 16, body, 0)

            @pl.when(cid == 0)
            def _wr():
                pltpu.sync_copy(r_ref, d_hbm.at[pl.ds(base, TPT)])

        dest_half(e0_v, r0_v, d0_hbm)
        dest_half(e1_v, r1_v, d1_hbm)

        @pl.when(jnp.logical_and(sid == 0, cid == 0))
        def _be():
            for j in range(NB_PAD // 16):
                bvec = lax.iota(jnp.int32, 16) + (j * 16)
                be = jnp.zeros((16,), jnp.int32)
                for e in range(1, E):
                    be = be + jnp.where(bvec >= bstart[e], 1, 0)
                be_v[pl.ds(j * 16, 16)] = be
            pltpu.sync_copy(be_v, be_hbm)

        tbase = base + cid * TPW
        off = cid * TPW

        def chunk_body(c, _):
            slot = lax.rem(c, 2)
            pltpu.sync_copy(x_hbm.at[pl.ds(tbase + c * CS, CS)],
                            rows2_v.at[slot])
            a = pltpu.async_copy(
                rows2_v.at[slot],
                xp_hbm.at[r0_v.at[pl.ds(off + c * CS, CS)]], sem0)
            b = pltpu.async_copy(
                rows2_v.at[slot],
                xp_hbm.at[r1_v.at[pl.ds(off + c * CS, CS)]], sem1)
            a.wait()
            b.wait()
            return 0

        lax.fori_loop(0, TPW // CS, chunk_body, 0)

    return k(i0, i1, moe_inp)


def _sc_combine(y_pad, dest0, dest1, s0, s1):
    mesh = plsc.VectorSubcoreMesh(core_axis_name="c", subcore_axis_name="s")

    nchunk = TPW // C2

    @functools.partial(
        pl.kernel, mesh=mesh,
        out_type=jax.ShapeDtypeStruct((T, D), jnp.float32),
        scratch_types=[
            pltpu.VMEM((TPW,), jnp.int32),
            pltpu.VMEM((TPW,), jnp.int32),
            pltpu.VMEM((TPW,), jnp.float32),
            pltpu.VMEM((TPW,), jnp.float32),
            pltpu.VMEM((2, C2, D), jnp.float32),
            pltpu.VMEM((2, C2, D), jnp.float32),
            pltpu.SemaphoreType.DMA,
            pltpu.SemaphoreType.DMA,
            pltpu.SemaphoreType.DMA,
        ],
    )
    def k(y_hbm, d0_hbm, d1_hbm, s0_hbm, s1_hbm, out_hbm,
          i0_v, i1_v, s0_v, s1_v, buf0, buf1, gsem0, gsem1, osem):
        wid = lax.axis_index("s") * NC + lax.axis_index("c")
        base = wid * TPW
        pltpu.sync_copy(d0_hbm.at[pl.ds(base, TPW)], i0_v)
        pltpu.sync_copy(d1_hbm.at[pl.ds(base, TPW)], i1_v)
        pltpu.sync_copy(s0_hbm.at[pl.ds(base, TPW)], s0_v)
        pltpu.sync_copy(s1_hbm.at[pl.ds(base, TPW)], s1_v)

        def gather(c, slot):
            pltpu.async_copy(y_hbm.at[i0_v.at[pl.ds(c * C2, C2)]],
                             buf0.at[slot], gsem0)
            pltpu.async_copy(y_hbm.at[i1_v.at[pl.ds(c * C2, C2)]],
                             buf1.at[slot], gsem1)

        def wait_gather(c, slot):
            pltpu.make_async_copy(y_hbm.at[i0_v.at[pl.ds(c * C2, C2)]],
                                  buf0.at[slot], gsem0).wait()
            pltpu.make_async_copy(y_hbm.at[i1_v.at[pl.ds(c * C2, C2)]],
                                  buf1.at[slot], gsem1).wait()

        def wait_out(c, slot):
            pltpu.make_async_copy(
                buf0.at[slot],
                out_hbm.at[pl.ds(base + c * C2, C2)], osem).wait()

        gather(0, 0)

        def chunk_body(c, _):
            slot = lax.rem(c, 2)
            wait_gather(c, slot)

            @pl.when(c + 1 < nchunk)
            def _pref():
                gather(c + 1, lax.rem(c + 1, 2))

            @pl.when(c >= 2)
            def _owait():
                wait_out(c - 2, slot)

            for g in range(C2 // 16):
                svec0 = s0_v[pl.ds(c * C2 + g * 16, 16)]
                svec1 = s1_v[pl.ds(c * C2 + g * 16, 16)]
                for r16 in range(16):
                    r = g * 16 + r16
                    sb0 = svec0[r16]
                    sb1 = svec1[r16]

                    def jbody(j, _, r=r, sb0=sb0, sb1=sb1):
                        sl = pl.ds(j * 16, 16)
                        buf0[slot, r, sl] = (sb0 * buf0[slot, r, sl]
                                             + sb1 * buf1[slot, r, sl])
                        return 0

                    lax.fori_loop(0, D // 16, jbody, 0, unroll=8)
            pltpu.async_copy(buf0.at[slot],
                             out_hbm.at[pl.ds(base + c * C2, C2)], osem)
            return 0

        lax.fori_loop(0, nchunk, chunk_body, 0)
        wait_out(nchunk - 2, lax.rem(nchunk - 2, 2))
        wait_out(nchunk - 1, lax.rem(nchunk - 1, 2))

    return k(y_pad, dest0, dest1, s0, s1)


def kernel(moe_inp, attn_weights, gate_w, gate_b, w1, b1, w2, b2):
    del attn_weights
    i0, i1, s0, s1 = _gating(moe_inp, gate_w, gate_b)
    i0, i1, s0, s1 = i0[0], i1[0], s0[0], s1[0]

    x_pad, dest0, dest1, block_expert = _sc_route_scatter(i0, i1, moe_inp)
    y_pad = _grouped_mlp(block_expert, x_pad, w1, b1, w2, b2)
    out = _sc_combine(y_pad, dest0, dest1, s0, s1)
    return out
